# baseline (speedup 1.0000x reference)
import functools

import jax
import jax.numpy as jnp
from jax.experimental import pallas as pl
from jax.experimental.pallas import tpu as pltpu

LEVELS = 4
K = 1024
D = 256
BETA = 0.25
HID = 512
N = 8192
BT = 512


def _stream_kernel(ind,
                   x_ref, cb_ref,
                   eW1_ref, eb1_ref, eW2_ref, eb2_ref,
                   dW1_ref, db1_ref, dW2_ref, db2_ref,
                   recon_ref, idx_ref, loss_ref):
    pid = pl.program_id(0)
    x = x_ref[...]

    h = jnp.maximum(jnp.dot(x, eW1_ref[...]) + eb1_ref[...], 0.0)
    z = jnp.dot(h, eW2_ref[...]) + eb2_ref[...]

    residual = z
    q_total = jnp.zeros_like(z)
    loss_sq = jnp.zeros((), dtype=jnp.float32)
    idx_cols = []
    bt = z.shape[0]
    for l in range(LEVELS):
        cb = cb_ref[l]
        r2 = jnp.sum(residual * residual, axis=-1, keepdims=True)
        s = jax.lax.dot_general(residual, cb,
                                (((1,), (1,)), ((), ())))
        csq = jnp.sum(cb * cb, axis=-1)
        d = r2 - 2.0 * s + csq[None, :]
        m = jnp.min(d, axis=-1, keepdims=True)
        iota = jax.lax.broadcasted_iota(jnp.int32, d.shape, 1)
        idx = jnp.min(jnp.where(d == m, iota, K), axis=-1)
        g = jax.lax.shift_right_logical(idx, 3)
        iota128 = jax.lax.broadcasted_iota(jnp.int32, (bt, K // 8), 1)
        oh = (iota128 == g[:, None]).astype(jnp.float32)
        grp = jnp.dot(oh, cb.reshape(K // 8, 8 * D),
                      precision=jax.lax.Precision.HIGHEST)
        sub = jnp.broadcast_to((idx & 7)[:, None, None], (bt, 1, D))
        q_l = jnp.take_along_axis(grp.reshape(bt, 8, D), sub,
                                  axis=1).reshape(bt, D)
        loss_sq = loss_sq + jnp.sum((residual - q_l) ** 2)
        residual = residual - q_l
        q_total = q_total + q_l
        idx_cols.append(idx)

    q_st = z + (q_total - z)

    hd = jnp.maximum(jnp.dot(q_st, dW1_ref[...]) + db1_ref[...], 0.0)
    recon = jnp.dot(hd, dW2_ref[...]) + db2_ref[...]
    recon_ref[...] = recon

    for l in range(LEVELS):
        idx_ref[:, l] = idx_cols[l]

    recon_sq = jnp.sum((recon - x) ** 2)
    acc = ((1.0 + BETA) / (N * D)) * loss_sq + (1.0 / (N * ind)) * recon_sq

    @pl.when(pid == 0)
    def _():
        loss_ref[...] = jnp.zeros_like(loss_ref)

    loss_ref[...] = loss_ref[...] + acc


def _run_stream(x, codebooks, eW1, eb1, eW2, eb2, dW1, db1, dW2, db2):
    n, ind = x.shape
    grid = (n // BT,)
    full = lambda shape: pl.BlockSpec(shape, lambda i: (0,) * len(shape))
    recon, idx, loss = pl.pallas_call(
        functools.partial(_stream_kernel, ind),
        grid=grid,
        in_specs=[
            pl.BlockSpec((BT, ind), lambda i: (i, 0)),
            full((LEVELS, K, D)),
            full((ind, HID)), full((1, HID)),
            full((HID, D)), full((1, D)),
            full((D, HID)), full((1, HID)),
            full((HID, ind)), full((1, ind)),
        ],
        out_specs=[
            pl.BlockSpec((BT, ind), lambda i: (i, 0)),
            pl.BlockSpec((BT, LEVELS), lambda i: (i, 0)),
            pl.BlockSpec((8, 128), lambda i: (0, 0)),
        ],
        out_shape=[
            jax.ShapeDtypeStruct((n, ind), jnp.float32),
            jax.ShapeDtypeStruct((n, LEVELS), jnp.int32),
            jax.ShapeDtypeStruct((8, 128), jnp.float32),
        ],
        compiler_params=pltpu.CompilerParams(
            dimension_semantics=("arbitrary",),
        ),
    )(x, codebooks,
      eW1, eb1.reshape(1, HID), eW2, eb2.reshape(1, D),
      dW1, db1.reshape(1, HID), dW2, db2.reshape(1, ind))
    codes = [idx[:, l] for l in range(LEVELS)]
    return recon, codes, loss[0, 0]


def kernel(body, hands, codebooks,
           enc_body_W1, enc_body_b1, enc_body_W2, enc_body_b2,
           dec_body_W1, dec_body_b1, dec_body_W2, dec_body_b2,
           enc_hands_W1, enc_hands_b1, enc_hands_W2, enc_hands_b2,
           dec_hands_W1, dec_hands_b1, dec_hands_W2, dec_hands_b2):
    recon_b, codes_b, loss_b = _run_stream(
        body, codebooks, enc_body_W1, enc_body_b1, enc_body_W2, enc_body_b2,
        dec_body_W1, dec_body_b1, dec_body_W2, dec_body_b2)
    recon_h, codes_h, loss_h = _run_stream(
        hands, codebooks, enc_hands_W1, enc_hands_b1, enc_hands_W2,
        enc_hands_b2, dec_hands_W1, dec_hands_b1, dec_hands_W2, dec_hands_b2)
    outputs = {"body": recon_b, "hands": recon_h}
    codes = {"body": codes_b, "hands": codes_h}
    total = loss_b + loss_h
    return outputs, codes, total


# speedup vs baseline: 2.6518x; 2.6518x over previous
import functools

import jax
import jax.numpy as jnp
from jax.experimental import pallas as pl
from jax.experimental.pallas import tpu as pltpu

LEVELS = 4
K = 1024
D = 256
BETA = 0.25
HID = 512
N = 8192
BT = 512


def _stream_kernel(ind,
                   x_ref, cb_ref,
                   eW1_ref, eb1_ref, eW2_ref, eb2_ref,
                   dW1_ref, db1_ref, dW2_ref, db2_ref,
                   recon_ref, idx_ref, loss_ref):
    pid = pl.program_id(0)
    x = x_ref[...]

    h = jnp.maximum(jnp.dot(x, eW1_ref[...]) + eb1_ref[...], 0.0)
    z = jnp.dot(h, eW2_ref[...]) + eb2_ref[...]

    residual = z
    q_total = jnp.zeros_like(z)
    loss_sq = jnp.zeros((), dtype=jnp.float32)
    idx_cols = []
    bt = z.shape[0]
    for l in range(LEVELS):
        cb = cb_ref[l]
        r2 = jnp.sum(residual * residual, axis=-1, keepdims=True)
        s = jax.lax.dot_general(residual, cb,
                                (((1,), (1,)), ((), ())))
        csq = jnp.sum(cb * cb, axis=-1)
        d = r2 - 2.0 * s + csq[None, :]
        m = jnp.min(d, axis=-1, keepdims=True)
        iota = jax.lax.broadcasted_iota(jnp.int32, d.shape, 1)
        idx = jnp.min(jnp.where(d == m, iota, K), axis=-1)
        oh = (iota == idx[:, None]).astype(jnp.float32)
        q_l = jnp.dot(oh, cb)
        loss_sq = loss_sq + jnp.sum((residual - q_l) ** 2)
        residual = residual - q_l
        q_total = q_total + q_l
        idx_cols.append(idx)

    q_st = z + (q_total - z)

    hd = jnp.maximum(jnp.dot(q_st, dW1_ref[...]) + db1_ref[...], 0.0)
    recon = jnp.dot(hd, dW2_ref[...]) + db2_ref[...]
    recon_ref[...] = recon

    for l in range(LEVELS):
        idx_ref[:, l] = idx_cols[l]

    recon_sq = jnp.sum((recon - x) ** 2)
    acc = ((1.0 + BETA) / (N * D)) * loss_sq + (1.0 / (N * ind)) * recon_sq

    @pl.when(pid == 0)
    def _():
        loss_ref[...] = jnp.zeros_like(loss_ref)

    loss_ref[...] = loss_ref[...] + acc


def _run_stream(x, codebooks, eW1, eb1, eW2, eb2, dW1, db1, dW2, db2):
    n, ind = x.shape
    grid = (n // BT,)
    full = lambda shape: pl.BlockSpec(shape, lambda i: (0,) * len(shape))
    recon, idx, loss = pl.pallas_call(
        functools.partial(_stream_kernel, ind),
        grid=grid,
        in_specs=[
            pl.BlockSpec((BT, ind), lambda i: (i, 0)),
            full((LEVELS, K, D)),
            full((ind, HID)), full((1, HID)),
            full((HID, D)), full((1, D)),
            full((D, HID)), full((1, HID)),
            full((HID, ind)), full((1, ind)),
        ],
        out_specs=[
            pl.BlockSpec((BT, ind), lambda i: (i, 0)),
            pl.BlockSpec((BT, LEVELS), lambda i: (i, 0)),
            pl.BlockSpec((8, 128), lambda i: (0, 0)),
        ],
        out_shape=[
            jax.ShapeDtypeStruct((n, ind), jnp.float32),
            jax.ShapeDtypeStruct((n, LEVELS), jnp.int32),
            jax.ShapeDtypeStruct((8, 128), jnp.float32),
        ],
        compiler_params=pltpu.CompilerParams(
            dimension_semantics=("arbitrary",),
        ),
    )(x, codebooks,
      eW1, eb1.reshape(1, HID), eW2, eb2.reshape(1, D),
      dW1, db1.reshape(1, HID), dW2, db2.reshape(1, ind))
    codes = [idx[:, l] for l in range(LEVELS)]
    return recon, codes, loss[0, 0]


def kernel(body, hands, codebooks,
           enc_body_W1, enc_body_b1, enc_body_W2, enc_body_b2,
           dec_body_W1, dec_body_b1, dec_body_W2, dec_body_b2,
           enc_hands_W1, enc_hands_b1, enc_hands_W2, enc_hands_b2,
           dec_hands_W1, dec_hands_b1, dec_hands_W2, dec_hands_b2):
    recon_b, codes_b, loss_b = _run_stream(
        body, codebooks, enc_body_W1, enc_body_b1, enc_body_W2, enc_body_b2,
        dec_body_W1, dec_body_b1, dec_body_W2, dec_body_b2)
    recon_h, codes_h, loss_h = _run_stream(
        hands, codebooks, enc_hands_W1, enc_hands_b1, enc_hands_W2,
        enc_hands_b2, dec_hands_W1, dec_hands_b1, dec_hands_W2, dec_hands_b2)
    outputs = {"body": recon_b, "hands": recon_h}
    codes = {"body": codes_b, "hands": codes_h}
    total = loss_b + loss_h
    return outputs, codes, total
